# Initial kernel scaffold; baseline (speedup 1.0000x reference)
import jax
import jax.numpy as jnp
from jax.experimental import pallas as pl


def kernel(ids, cb1, cb2, cb3, W_dec, b_dec):
    raise NotImplementedError("write your pallas kernel here")


# speedup vs baseline: 3.5541x; 3.5541x over previous
import functools

import jax
import jax.numpy as jnp
from jax import lax
from jax.experimental import pallas as pl
from jax.experimental.pallas import tpu as pltpu
from jax.experimental.pallas import tpu_sc as plsc

B = 16
T = 1024
K = 4096
D = 64
HOP = 128
C = B * T
F = 4 * C

NC = 2
NS = 16
NW = NC * NS
C_W = C // NW
F_W = 4 * C_W
NSUB = 4
C_SUB = C_W // NSUB
F_SUB = 4 * C_SUB
SEG = 128
NSEG = F_SUB // SEG


def _sc_gather_combine(ids_flat, cb1, cb2, cb3):
    mesh = plsc.VectorSubcoreMesh(core_axis_name="c", subcore_axis_name="s")

    @functools.partial(
        pl.kernel,
        out_type=jax.ShapeDtypeStruct((F, D), jnp.float32),
        mesh=mesh,
        scratch_types=[
            pltpu.VMEM((C_W * 7,), jnp.int32),
            pltpu.VMEM((NSEG, SEG), jnp.int32),
            pltpu.VMEM((NSEG, SEG), jnp.int32),
            pltpu.VMEM((NSEG, SEG), jnp.int32),
            pltpu.VMEM((F_SUB, D), jnp.float32),
            pltpu.SemaphoreType.DMA,
        ],
        compiler_params=pltpu.CompilerParams(needs_layout_passes=False,
                                             use_tc_tiling_on_sc=False),
    )
    def k(ids_hbm, cb1_hbm, cb2_hbm, cb3_hbm, z_hbm,
          ids_v, idx1_v, idx2_v, idx3_v, z_v, sem):
        wid = lax.axis_index("s") * NC + lax.axis_index("c")
        base_c = wid * C_W
        pltpu.sync_copy(ids_hbm.at[pl.ds(base_c * 7, C_W * 7)], ids_v)
        lane = lax.broadcasted_iota(jnp.int32, (16,), 0)
        for sub in range(NSUB):
            for i in range(F_SUB // 16):
                f = lane + (i * 16 + sub * F_SUB)
                t = f >> 2
                s = f & 3
                base7 = t * 7
                v1 = plsc.load_gather(ids_v, [base7])
                v2 = plsc.load_gather(ids_v, [base7 + (1 + (s >> 1))]) - K
                v3 = plsc.load_gather(ids_v, [base7 + (3 + s)]) - 2 * K
                seg, off = divmod(i * 16, SEG)
                idx1_v[seg, pl.ds(off, 16)] = v1
                idx2_v[seg, pl.ds(off, 16)] = v2
                idx3_v[seg, pl.ds(off, 16)] = v3
            cps = [pltpu.async_copy(cb3_hbm.at[idx3_v.at[g]],
                                    z_v.at[pl.ds(g * SEG, SEG)], sem)
                   for g in range(NSEG)]
            for cp in cps:
                cp.wait()
            cps = [pltpu.async_copy(cb2_hbm.at[idx2_v.at[g]],
                                    z_v.at[pl.ds(g * SEG, SEG)], sem, add=True)
                   for g in range(NSEG)]
            for cp in cps:
                cp.wait()
            cps = [pltpu.async_copy(cb1_hbm.at[idx1_v.at[g]],
                                    z_v.at[pl.ds(g * SEG, SEG)], sem, add=True)
                   for g in range(NSEG)]
            for cp in cps:
                cp.wait()
            base_f = wid * F_W + sub * F_SUB
            pltpu.sync_copy(z_v, z_hbm.at[pl.ds(base_f, F_SUB)])

    return k(ids_flat, cb1, cb2, cb3)


def _tc_decode(z, W_dec, b_dec):
    ROWS = 1024

    def body(z_ref, w_ref, b_ref, o_ref):
        acc = jnp.dot(z_ref[...], w_ref[...],
                      preferred_element_type=jnp.float32)
        o_ref[...] = jnp.tanh(acc + b_ref[...])

    return pl.pallas_call(
        body,
        grid=(F // ROWS,),
        in_specs=[
            pl.BlockSpec((ROWS, D), lambda i: (i, 0)),
            pl.BlockSpec((D, HOP), lambda i: (0, 0)),
            pl.BlockSpec((1, HOP), lambda i: (0, 0)),
        ],
        out_specs=pl.BlockSpec((ROWS, HOP), lambda i: (i, 0)),
        out_shape=jax.ShapeDtypeStruct((F, HOP), jnp.float32),
    )(z, W_dec, b_dec.reshape(1, HOP))


def kernel(ids, cb1, cb2, cb3, W_dec, b_dec):
    ids_flat = ids.reshape(-1).astype(jnp.int32)
    z = _sc_gather_combine(ids_flat, cb1, cb2, cb3)
    out = _tc_decode(z, W_dec, b_dec)
    return out.reshape(B, 1, 4 * T * HOP)


# speedup vs baseline: 5.5221x; 1.5537x over previous
import functools

import jax
import jax.numpy as jnp
from jax import lax
from jax.experimental import pallas as pl
from jax.experimental.pallas import tpu as pltpu
from jax.experimental.pallas import tpu_sc as plsc

B = 16
T = 1024
K = 4096
D = 64
HOP = 128
C = B * T
F = 4 * C

NC = 2
NS = 16
NW = NC * NS
C_W = C // NW
F_W = 4 * C_W
NSUB = 4
C_SUB = C_W // NSUB
F_SUB = 4 * C_SUB
SEG = 128
NSEG = F_SUB // SEG


def _sc_gather_combine(ids_flat, cb1, cb2, cb3):
    mesh = plsc.VectorSubcoreMesh(core_axis_name="c", subcore_axis_name="s")

    @functools.partial(
        pl.kernel,
        out_type=jax.ShapeDtypeStruct((F, 2 * D), jnp.float32),
        mesh=mesh,
        scratch_types=[
            pltpu.VMEM((C_W * 7,), jnp.int32),
            pltpu.VMEM((NSEG, SEG), jnp.int32),
            pltpu.VMEM((NSEG, SEG), jnp.int32),
            pltpu.VMEM((NSEG, SEG), jnp.int32),
            pltpu.VMEM((F_SUB, D), jnp.float32),
            pltpu.SemaphoreType.DMA,
        ],
        compiler_params=pltpu.CompilerParams(needs_layout_passes=False,
                                             use_tc_tiling_on_sc=False),
    )
    def k(ids_hbm, cb1_hbm, cb2_hbm, cb3_hbm, z_hbm,
          ids_v, idx1_v, idx2_v, idx3_v, z_v, sem):
        wid = lax.axis_index("s") * NC + lax.axis_index("c")
        base_c = wid * C_W
        pltpu.sync_copy(ids_hbm.at[pl.ds(base_c * 7, C_W * 7)], ids_v)
        lane = lax.broadcasted_iota(jnp.int32, (16,), 0)
        for sub in range(NSUB):
            for i in range(F_SUB // 16):
                f = lane + (i * 16 + sub * F_SUB)
                t = f >> 2
                s = f & 3
                base7 = t * 7
                v1 = plsc.load_gather(ids_v, [base7])
                v2 = plsc.load_gather(ids_v, [base7 + (1 + (s >> 1))]) - K
                v3 = plsc.load_gather(ids_v, [base7 + (3 + s)]) - 2 * K
                seg, off = divmod(i * 16, SEG)
                idx1_v[seg, pl.ds(off, 16)] = v1
                idx2_v[seg, pl.ds(off, 16)] = v2
                idx3_v[seg, pl.ds(off, 16)] = v3
            cps = [pltpu.async_copy(cb3_hbm.at[idx3_v.at[g]],
                                    z_v.at[pl.ds(g * SEG, SEG)], sem)
                   for g in range(NSEG)]
            for cp in cps:
                cp.wait()
            cps = [pltpu.async_copy(cb2_hbm.at[idx2_v.at[g]],
                                    z_v.at[pl.ds(g * SEG, SEG)], sem, add=True)
                   for g in range(NSEG)]
            for cp in cps:
                cp.wait()
            cps = [pltpu.async_copy(cb1_hbm.at[idx1_v.at[g]],
                                    z_v.at[pl.ds(g * SEG, SEG)], sem, add=True)
                   for g in range(NSEG)]
            for cp in cps:
                cp.wait()
            base_f = wid * F_W + sub * F_SUB
            pltpu.sync_copy(z_v, z_hbm.at[pl.ds(base_f, F_SUB), pl.ds(0, D)])

    return k(ids_flat, cb1, cb2, cb3)


def _tc_decode(z, W_dec, b_dec):
    ROWS = 4096

    def body(z_ref, w_ref, b_ref, o_ref):
        acc = jnp.dot(z_ref[:, :D], w_ref[...],
                      preferred_element_type=jnp.float32)
        o_ref[...] = jnp.tanh(acc + b_ref[...])

    return pl.pallas_call(
        body,
        grid=(F // ROWS,),
        in_specs=[
            pl.BlockSpec((ROWS, 2 * D), lambda i: (i, 0)),
            pl.BlockSpec((D, HOP), lambda i: (0, 0)),
            pl.BlockSpec((1, HOP), lambda i: (0, 0)),
        ],
        out_specs=pl.BlockSpec((ROWS, HOP), lambda i: (i, 0)),
        out_shape=jax.ShapeDtypeStruct((F, HOP), jnp.float32),
    )(z, W_dec, b_dec.reshape(1, HOP))


def kernel(ids, cb1, cb2, cb3, W_dec, b_dec):
    ids_flat = ids.reshape(-1).astype(jnp.int32)
    z = _sc_gather_combine(ids_flat, cb1, cb2, cb3)
    out = _tc_decode(z, W_dec, b_dec)
    return out.reshape(B, 1, 4 * T * HOP)
